# baseline (speedup 1.0000x reference)
import jax
import jax.numpy as jnp
from jax.experimental import pallas as pl

B, N1, N2 = 8, 1024, 8192
C1, C2 = 256, 128
OUTC = 3 + C1 + C2
PADC = 512
TILE = 4096


def _fused_body(x0_ref, x1_ref, out_ref):
    x0b = x0_ref[0]
    xyz1 = x0b[:, 0:3]
    f1 = x0b[:, 3:].astype(jnp.bfloat16)
    x1b = x1_ref[0]
    xyz2 = x1b[:, 0:3]
    f2 = x1b[:, 3:]

    cross = jax.lax.dot_general(
        xyz2, xyz1, (((1,), (1,)), ((), ())),
        preferred_element_type=jnp.float32)
    x2sq = jnp.sum(xyz2 * xyz2, axis=1, keepdims=True)
    x1sq = jnp.sum(xyz1 * xyz1, axis=1, keepdims=True).T
    d = x2sq - 2.0 * cross + x1sq
    idx = jnp.argmin(d, axis=1)

    onehot = (jax.lax.broadcasted_iota(jnp.int32, (TILE, N1), 1)
              == idx[:, None]).astype(jnp.bfloat16)
    nearest = jnp.dot(onehot, f1, preferred_element_type=jnp.float32)

    out_ref[0, :, 0:3] = xyz2
    out_ref[0, :, 3:3 + C1] = nearest
    out_ref[0, :, 3 + C1:OUTC] = f2
    out_ref[0, :, OUTC:] = jnp.zeros((TILE, PADC - OUTC), jnp.float32)


def kernel(x0, x1):
    outp = pl.pallas_call(
        _fused_body,
        grid=(B, N2 // TILE),
        in_specs=[
            pl.BlockSpec((1, N1, 259), lambda b, t: (b, 0, 0)),
            pl.BlockSpec((1, TILE, 3 + C2), lambda b, t: (b, t, 0)),
        ],
        out_specs=pl.BlockSpec((1, TILE, PADC), lambda b, t: (b, t, 0)),
        out_shape=jax.ShapeDtypeStruct((B, N2, PADC), jnp.float32),
    )(x0, x1)
    return (outp[:, :, :OUTC], x1[:, :, 0:3])


# speedup vs baseline: 1.1500x; 1.1500x over previous
import jax
import jax.numpy as jnp
from jax.experimental import pallas as pl

B, N1, N2 = 8, 1024, 8192
C1, C2 = 256, 128
OUTC = 3 + C1 + C2
PADC = 512
TILE = 4096


def _fused_body(xyz1t_ref, f1_ref, x1_ref, out_ref):
    xyz1t = xyz1t_ref[0]
    f1 = f1_ref[0]
    x1b = x1_ref[0]
    xyz2 = x1b[:, 0:3]
    f2 = x1b[:, 3:]

    cross = jnp.dot(xyz2, xyz1t, preferred_element_type=jnp.float32)
    x2sq = jnp.sum(xyz2 * xyz2, axis=1, keepdims=True)
    x1sq = jnp.sum(xyz1t * xyz1t, axis=0, keepdims=True)
    d = x2sq - 2.0 * cross + x1sq
    idx = jnp.argmin(d, axis=1)

    onehot = (jax.lax.broadcasted_iota(jnp.int32, (TILE, N1), 1)
              == idx[:, None]).astype(jnp.bfloat16)
    nearest = jnp.dot(onehot, f1, preferred_element_type=jnp.float32)

    out_ref[0, :, 0:3] = xyz2
    out_ref[0, :, 3:3 + C1] = nearest
    out_ref[0, :, 3 + C1:OUTC] = f2
    out_ref[0, :, OUTC:] = jnp.zeros((TILE, PADC - OUTC), jnp.float32)


def kernel(x0, x1):
    xyz1t = jnp.transpose(x0[:, :, 0:3], (0, 2, 1))
    f1 = x0[:, :, 3:].astype(jnp.bfloat16)
    out = pl.pallas_call(
        _fused_body,
        grid=(B, N2 // TILE),
        in_specs=[
            pl.BlockSpec((1, 3, N1), lambda b, t: (b, 0, 0)),
            pl.BlockSpec((1, N1, C1), lambda b, t: (b, 0, 0)),
            pl.BlockSpec((1, TILE, 3 + C2), lambda b, t: (b, t, 0)),
        ],
        out_specs=pl.BlockSpec((1, TILE, PADC), lambda b, t: (b, t, 0)),
        out_shape=jax.ShapeDtypeStruct((B, N2, PADC), jnp.float32),
    )(xyz1t, f1, x1)
    return (out[:, :, :OUTC], x1[:, :, 0:3])


# speedup vs baseline: 1.1503x; 1.0003x over previous
import jax
import jax.numpy as jnp
from jax.experimental import pallas as pl

B, N1, N2 = 8, 1024, 8192
C1, C2 = 256, 128
OUTC = 3 + C1 + C2
PADC = 512
TILE = 4096


def _fused_body(xyz1t_ref, f1_ref, x1_ref, out_ref):
    xyz1t = xyz1t_ref[0][0:3]
    f1 = f1_ref[0]
    x1b = x1_ref[0]
    xyz2 = x1b[:, 0:3]
    f2 = x1b[:, 3:]

    cross = jnp.dot(xyz2, xyz1t, preferred_element_type=jnp.float32)
    x2sq = jnp.sum(xyz2 * xyz2, axis=1, keepdims=True)
    x1sq = jnp.sum(xyz1t * xyz1t, axis=0, keepdims=True)
    d = x2sq - 2.0 * cross + x1sq
    idx = jnp.argmin(d, axis=1)

    onehot = (jax.lax.broadcasted_iota(jnp.int32, (TILE, N1), 1)
              == idx[:, None]).astype(jnp.bfloat16)
    nearest = jnp.dot(onehot, f1, preferred_element_type=jnp.float32)

    out_ref[0, :, 0:3] = xyz2
    out_ref[0, :, 3:3 + C1] = nearest
    out_ref[0, :, 3 + C1:OUTC] = f2
    out_ref[0, :, OUTC:] = jnp.zeros((TILE, PADC - OUTC), jnp.float32)


def kernel(x0, x1):
    xyz1t = jnp.pad(jnp.transpose(x0[:, :, 0:3], (0, 2, 1)),
                    ((0, 0), (0, 5), (0, 0)))
    f1 = x0[:, :, 3:].astype(jnp.bfloat16)
    out = pl.pallas_call(
        _fused_body,
        grid=(B, N2 // TILE),
        in_specs=[
            pl.BlockSpec((1, 8, N1), lambda b, t: (b, 0, 0)),
            pl.BlockSpec((1, N1, C1), lambda b, t: (b, 0, 0)),
            pl.BlockSpec((1, TILE, 3 + C2), lambda b, t: (b, t, 0)),
        ],
        out_specs=pl.BlockSpec((1, TILE, PADC), lambda b, t: (b, t, 0)),
        out_shape=jax.ShapeDtypeStruct((B, N2, PADC), jnp.float32),
    )(xyz1t, f1, x1)
    return (out[:, :, :OUTC], x1[:, :, 0:3])
